# Initial kernel scaffold; baseline (speedup 1.0000x reference)
import jax
import jax.numpy as jnp
from jax.experimental import pallas as pl


def kernel(features, mean_vertices, edge_index, W_enc1, b_enc1, W_enc2, b_enc2, W1, b1, W2, b2, W3, b3):
    raise NotImplementedError("write your pallas kernel here")


# speedup vs baseline: 44.9288x; 44.9288x over previous
import functools

import jax
import jax.numpy as jnp
from jax import lax
from jax.experimental import pallas as pl
from jax.experimental.pallas import tpu as pltpu
from jax.experimental.pallas import tpu_sc as plsc

_NC, _NS, _LANES = 2, 16, 16
_NW = _NC * _NS
_CH = 128


def _chunk_offsets(total, chunk):
  offs = list(range(0, total - chunk + 1, chunk))
  if offs[-1] != total - chunk:
    offs.append(total - chunk)
  return offs


@functools.lru_cache(maxsize=None)
def _sc_degree_fn(NP1, NCH):
  CPT = NP1 // _NS
  assert CPT % 128 == 0 and _NS * CPT == NP1
  mesh = plsc.VectorSubcoreMesh(core_axis_name="c", subcore_axis_name="s",
                                num_cores=_NC, num_subcores=_NS)

  @functools.partial(
      pl.kernel,
      out_type=jax.ShapeDtypeStruct((_NC, NP1), jnp.float32),
      mesh=mesh,
      scratch_types=[
          pltpu.VMEM((NCH, _CH), jnp.int32),
          pltpu.VMEM((_CH,), jnp.float32),
          pltpu.VMEM((CPT,), jnp.float32),
          pltpu.VMEM_SHARED((NP1,), jnp.float32),
      ],
  )
  def body(dst_hbm, ones_hbm, zer_hbm, out_hbm, dst_v, ones_v, dbuf, acc):
    c = lax.axis_index("c")
    s = lax.axis_index("s")
    w = c * _NS + s
    pltpu.sync_copy(dst_hbm.at[w], dst_v)
    pltpu.sync_copy(ones_hbm, ones_v)
    pltpu.sync_copy(zer_hbm, dbuf)
    pltpu.sync_copy(dbuf, acc.at[pl.ds(s * CPT, CPT)])
    plsc.subcore_barrier()

    def chunk(g, carry):
      pltpu.sync_copy(ones_v, acc.at[dst_v.at[g]], add=True)
      return carry

    lax.fori_loop(0, NCH, chunk, 0)
    plsc.subcore_barrier()
    pltpu.sync_copy(acc.at[pl.ds(s * CPT, CPT)], dbuf)
    pltpu.sync_copy(dbuf, out_hbm.at[c].at[pl.ds(s * CPT, CPT)])

  return body


@functools.lru_cache(maxsize=None)
def _sc_layer_fn(NP, NCH, D, NB):
  RPT = NP // _NS
  assert _NS * RPT == NP and RPT % 8 == 0
  offs = _chunk_offsets(RPT, _CH)
  ZCH = 64
  zoffs = _chunk_offsets(RPT, ZCH)
  mesh = plsc.VectorSubcoreMesh(core_axis_name="c", subcore_axis_name="s",
                                num_cores=_NC, num_subcores=_NS)

  @functools.partial(
      pl.kernel,
      out_type=jax.ShapeDtypeStruct((_NC * NB, NP, D), jnp.float32),
      mesh=mesh,
      scratch_types=[
          pltpu.VMEM((NCH, _CH), jnp.int32),
          pltpu.VMEM((NCH, _CH), jnp.int32),
          pltpu.VMEM((_CH, D), jnp.float32),
          pltpu.VMEM((ZCH, D), jnp.float32),
          pltpu.VMEM_SHARED((NP, D), jnp.float32),
      ],
  )
  def body(y_hbm, srcs_hbm, dst_hbm, zrow_hbm, out_hbm,
           src_v, dst_v, gbuf, zbuf, acc):
    c = lax.axis_index("c")
    s = lax.axis_index("s")
    w = c * _NS + s
    base = s * RPT
    pltpu.sync_copy(dst_hbm.at[w], dst_v)
    pltpu.sync_copy(zrow_hbm, zbuf)
    for b in range(NB):
      pltpu.sync_copy(srcs_hbm.at[b * _NW + w], src_v)
      for off in zoffs:
        pltpu.sync_copy(zbuf, acc.at[pl.ds(base + off, ZCH)])
      plsc.subcore_barrier()

      def chunk(g, carry):
        pltpu.sync_copy(y_hbm.at[src_v.at[g]], gbuf)
        pltpu.sync_copy(gbuf, acc.at[dst_v.at[g]], add=True)
        return carry

      lax.fori_loop(0, NCH, chunk, 0)
      plsc.subcore_barrier()
      slot = c * NB + b
      for off in offs:
        pltpu.sync_copy(acc.at[pl.ds(base + off, _CH)], gbuf)
        pltpu.sync_copy(gbuf, out_hbm.at[slot].at[pl.ds(base + off, _CH)])
      plsc.subcore_barrier()

  return body


def _tc_prep(f8, mv_pad, degt, W_enc1, b_enc1, W_enc2, b_enc2, w1a, w1b,
             N, NP, B, H, RB):
  NPB = NP // RB

  def body(f_ref, mv_ref, degt_ref, we1_ref, be1_ref, we2_ref, be2_ref,
           w1a_ref, w1b_ref, y_ref, dinv_ref):
    r = pl.program_id(0)
    degt_b = degt_ref[...]
    deg = degt_b[:, 0:1] + degt_b[:, 1:2] + 1.0
    riota = r * RB + lax.broadcasted_iota(jnp.int32, (RB, 1), 0)
    dinv = jnp.where(riota < N, lax.rsqrt(deg), 0.0)
    dinv_ref[...] = dinv
    f = f_ref[...]
    h = jax.nn.relu(
        jnp.dot(f, we1_ref[...], preferred_element_type=jnp.float32)
        + be1_ref[...][None, :])
    h = jax.nn.relu(
        jnp.dot(h, we2_ref[...], preferred_element_type=jnp.float32)
        + be2_ref[...][None, :])
    cmat = jnp.dot(h, w1a_ref[...], preferred_element_type=jnp.float32)
    w1b_v = w1b_ref[...]
    for b in range(B):
      mv_b = mv_ref[b]
      xw = (cmat[b][None, :]
            + mv_b[:, 0:1] * w1b_v[0][None, :]
            + mv_b[:, 1:2] * w1b_v[1][None, :]
            + mv_b[:, 2:3] * w1b_v[2][None, :])
      y_ref[b] = dinv * xw

  return pl.pallas_call(
      body,
      grid=(NPB,),
      in_specs=[
          pl.BlockSpec((8, f8.shape[1]), lambda r: (0, 0)),
          pl.BlockSpec((B, RB, 3), lambda r: (0, r, 0)),
          pl.BlockSpec((RB, 2), lambda r: (r, 0)),
          pl.BlockSpec(W_enc1.shape, lambda r: (0, 0)),
          pl.BlockSpec(b_enc1.shape, lambda r: (0,)),
          pl.BlockSpec(W_enc2.shape, lambda r: (0, 0)),
          pl.BlockSpec(b_enc2.shape, lambda r: (0,)),
          pl.BlockSpec(w1a.shape, lambda r: (0, 0)),
          pl.BlockSpec(w1b.shape, lambda r: (0, 0)),
      ],
      out_specs=[
          pl.BlockSpec((B, RB, H), lambda r: (0, r, 0)),
          pl.BlockSpec((RB, 1), lambda r: (r, 0)),
      ],
      out_shape=[
          jax.ShapeDtypeStruct((B, NP, H), jnp.float32),
          jax.ShapeDtypeStruct((NP, 1), jnp.float32),
      ],
  )(f8, mv_pad, degt, W_enc1, b_enc1, W_enc2, b_enc2, w1a, w1b)


def _tc_mid(p0, p1, yprev, dinv, bl, W, NP, B, H, RB):
  NPB = NP // RB

  def body(p0_ref, p1_ref, y_ref, dinv_ref, bl_ref, w_ref, out_ref):
    dv = dinv_ref[...]
    sagg = dv * (p0_ref[0] + p1_ref[0] + y_ref[0]) + bl_ref[...][None, :]
    o = jax.nn.relu(sagg)
    out_ref[0] = dv * jnp.dot(o, w_ref[...],
                              preferred_element_type=jnp.float32)

  return pl.pallas_call(
      body,
      grid=(B, NPB),
      in_specs=[
          pl.BlockSpec((1, RB, H), lambda b, r: (b, r, 0)),
          pl.BlockSpec((1, RB, H), lambda b, r: (b, r, 0)),
          pl.BlockSpec((1, RB, H), lambda b, r: (b, r, 0)),
          pl.BlockSpec((RB, 1), lambda b, r: (r, 0)),
          pl.BlockSpec(bl.shape, lambda b, r: (0,)),
          pl.BlockSpec(W.shape, lambda b, r: (0, 0)),
      ],
      out_specs=pl.BlockSpec((1, RB, H), lambda b, r: (b, r, 0)),
      out_shape=jax.ShapeDtypeStruct((B, NP, H), jnp.float32),
  )(p0, p1, yprev, dinv, bl, W)


def _tc_packw3(p0, p1, yprev, dinv, bl, W3, NP, B, H, RB):
  NPB = NP // RB

  def body(p0_ref, p1_ref, y_ref, dinv_ref, bl_ref, w3_ref, out_ref):
    dv = dinv_ref[...]
    sagg = dv * (p0_ref[0] + p1_ref[0] + y_ref[0]) + bl_ref[...][None, :]
    o = jax.nn.relu(sagg)
    t = jnp.dot(o, w3_ref[...], preferred_element_type=jnp.float32)
    out_ref[0] = jnp.concatenate(
        [dv * t, jnp.zeros((t.shape[0], 1), jnp.float32)], axis=1)

  return pl.pallas_call(
      body,
      grid=(B, NPB),
      in_specs=[
          pl.BlockSpec((1, RB, H), lambda b, r: (b, r, 0)),
          pl.BlockSpec((1, RB, H), lambda b, r: (b, r, 0)),
          pl.BlockSpec((1, RB, H), lambda b, r: (b, r, 0)),
          pl.BlockSpec((RB, 1), lambda b, r: (r, 0)),
          pl.BlockSpec(bl.shape, lambda b, r: (0,)),
          pl.BlockSpec(W3.shape, lambda b, r: (0, 0)),
      ],
      out_specs=pl.BlockSpec((1, RB, 4), lambda b, r: (b, r, 0)),
      out_shape=jax.ShapeDtypeStruct((B, NP, 4), jnp.float32),
  )(p0, p1, yprev, dinv, bl, W3)


def _tc_final(p30, p31, y3p, dinv, b3row, NP, RB):
  NPB = NP // RB

  def body(p0_ref, p1_ref, y_ref, dinv_ref, b3_ref, out_ref):
    dv = dinv_ref[...]
    u = dv * (p0_ref[...] + p1_ref[...] + y_ref[...]) + b3_ref[...][None, :]
    out_ref[...] = u[:, :16]

  return pl.pallas_call(
      body,
      grid=(NPB,),
      in_specs=[
          pl.BlockSpec((RB, 128), lambda r: (r, 0)),
          pl.BlockSpec((RB, 128), lambda r: (r, 0)),
          pl.BlockSpec((RB, 128), lambda r: (r, 0)),
          pl.BlockSpec((RB, 1), lambda r: (r, 0)),
          pl.BlockSpec((128,), lambda r: (0,)),
      ],
      out_specs=pl.BlockSpec((RB, 16), lambda r: (r, 0)),
      out_shape=jax.ShapeDtypeStruct((NP, 16), jnp.float32),
  )(p30, p31, y3p, dinv, b3row)


def kernel(features, mean_vertices, edge_index, W_enc1, b_enc1, W_enc2,
           b_enc2, W1, b1, W2, b2, W3, b3):
  B, FD = features.shape
  N = mean_vertices.shape[1]
  H = W2.shape[0]
  E = edge_index.shape[1]

  NP = ((N + 1 + 8 * _NS - 1) // (8 * _NS)) * (8 * _NS)
  NP1 = ((NP + 128 * _NS - 1) // (128 * _NS)) * (128 * _NS)
  EPT = ((E + _NW - 1) // _NW + _CH - 1) // _CH * _CH
  NCH = EPT // _CH
  ETOT = _NW * EPT
  RB = NP // 4
  assert RB % 8 == 0

  pad_n = ETOT - E
  padv = N + (jnp.arange(pad_n, dtype=jnp.int32) % (NP - N))
  srcp = jnp.concatenate([edge_index[0], padv]).reshape(_NW, NCH, _CH)
  dstp = jnp.concatenate([edge_index[1], padv]).reshape(_NW, NCH, _CH)
  boffs = (jnp.arange(B, dtype=jnp.int32) * NP)[:, None, None, None]
  srcs_wide = (srcp[None] + boffs).reshape(B * _NW, NCH, _CH)
  srcs_narrow = srcp.reshape(_NW, NCH, _CH)

  ones_ch = jnp.ones((_CH,), jnp.float32)
  zer_cpt = jnp.zeros((NP1 // _NS,), jnp.float32)
  zrow_wide = jnp.zeros((64, H), jnp.float32)

  f8 = jnp.zeros((8, FD), jnp.float32).at[:B].set(features)
  mv_pad = jnp.pad(mean_vertices, ((0, 0), (0, NP - N), (0, 0)))
  w1a = W1[:H]
  w1b = W1[H:H + 3]
  b3row = jnp.pad(
      jnp.tile(jnp.concatenate([b3, jnp.zeros((1,), jnp.float32)]), 4),
      (0, H - 16))

  deg_parts = _sc_degree_fn(NP1, NCH)(dstp, ones_ch, zer_cpt)
  degt = jnp.transpose(deg_parts)[:NP]

  y1, dinv = _tc_prep(f8, mv_pad, degt, W_enc1, b_enc1, W_enc2, b_enc2,
                      w1a, w1b, N, NP, B, H, RB)

  sc_wide = _sc_layer_fn(NP, NCH, H, B)
  out1 = sc_wide(y1.reshape(B * NP, H), srcs_wide, dstp, zrow_wide)
  y2 = _tc_mid(out1[:B], out1[B:], y1, dinv, b1, W2, NP, B, H, RB)

  out2 = sc_wide(y2.reshape(B * NP, H), srcs_wide, dstp, zrow_wide)
  y3t = _tc_packw3(out2[:B], out2[B:], y2, dinv, b2, W3, NP, B, H, RB)
  y3p = jnp.pad(jnp.transpose(y3t, (1, 0, 2)).reshape(NP, 4 * 4),
                ((0, 0), (0, H - 16)))

  out3 = _sc_layer_fn(NP, NCH, H, 1)(y3p, srcs_narrow, dstp, zrow_wide)
  u = _tc_final(out3[0], out3[1], y3p, dinv, b3row, NP, RB)

  delta = jnp.transpose(u[:N].reshape(N, 4, 4)[:, :, :3], (1, 0, 2))
  return delta
